# baseline (speedup 1.0000x reference)
import functools

import jax
import jax.numpy as jnp
from jax.experimental import pallas as pl
from jax.experimental.pallas import tpu as pltpu

_EPS = 1e-5
_TOPK = 2
_BLK = 512


def _dot_t(a, b, precision=None):
    return jax.lax.dot_general(
        a, b, dimension_numbers=(((1,), (1,)), ((), ())),
        precision=precision, preferred_element_type=jnp.float32)


def _silu(v):
    return v * jax.nn.sigmoid(v)


def _fused(G, EPG, H, x_ref, lnw_ref, lnb_ref, bso_ref, beo_ref, wg_ref,
           we_ref, gb_ref, eb_ref, wso_ref, wsi_hbm, wei_hbm, weo_hbm,
           out_ref, wsi_v, wei_v, weo_v, sem_si, sem_ei, sem_o1, sem_o2):
    E = G * EPG
    i = pl.program_id(0)

    @pl.when(i == 0)
    def _start_weight_dmas():
        pltpu.make_async_copy(wsi_hbm, wsi_v, sem_si).start()
        pltpu.make_async_copy(wei_hbm, wei_v, sem_ei).start()
        pltpu.make_async_copy(weo_hbm.at[:EPG], weo_v.at[:EPG], sem_o1).start()
        pltpu.make_async_copy(weo_hbm.at[EPG:], weo_v.at[EPG:], sem_o2).start()

    xb = x_ref[...]
    mu = jnp.mean(xb, axis=-1, keepdims=True)
    xc = xb - mu
    var = jnp.mean(xc * xc, axis=-1, keepdims=True)
    flat = xc / jnp.sqrt(var + _EPS) * lnw_ref[...] + lnb_ref[...]

    g = _dot_t(flat, wg_ref[...]) + gb_ref[...]
    el = _dot_t(flat, we_ref[...])
    eb = eb_ref[...]

    gmask = g[:, 1:2] > g[:, 0:1]
    e4 = (jnp.where(gmask, el[:, EPG:], el[:, :EPG])
          + jnp.where(gmask, eb[:, EPG:], eb[:, :EPG]))
    m = jnp.max(e4, axis=-1, keepdims=True)
    ex = jnp.exp(e4 - m)
    p = ex / jnp.sum(ex, axis=-1, keepdims=True)

    cols = [p[:, c_:c_ + 1] for c_ in range(EPG)]
    keep = []
    for e in range(EPG):
        cnt = jnp.zeros_like(cols[0])
        for j in range(EPG):
            if j == e:
                continue
            beats = (cols[j] >= cols[e]) if j < e else (cols[j] > cols[e])
            cnt = cnt + beats.astype(jnp.float32)
        keep.append((cnt < float(_TOPK)).astype(jnp.float32))
    c4 = p * jnp.concatenate(keep, axis=-1)
    gm = gmask.astype(jnp.float32)
    c8 = jnp.concatenate([c4 * (1.0 - gm), c4 * gm], axis=-1)

    @pl.when(i == 0)
    def _wait_shared_in():
        pltpu.make_async_copy(wsi_hbm, wsi_v, sem_si).wait()

    h_shared = (_silu(_dot_t(flat, wsi_v[:H, :])) * _dot_t(flat, wsi_v[H:, :]))
    acc = _dot_t(h_shared, wso_ref[...]) + bso_ref[...]

    @pl.when(i == 0)
    def _wait_expert_in():
        pltpu.make_async_copy(wei_hbm, wei_v, sem_ei).wait()

    h_expert = (_silu(_dot_t(flat, wei_v[:H, :])) * _dot_t(flat, wei_v[H:, :]))

    @pl.when(i == 0)
    def _wait_expert_out_lo():
        pltpu.make_async_copy(weo_hbm.at[:EPG], weo_v.at[:EPG], sem_o1).wait()

    for e in range(EPG):
        w = c8[:, e:e + 1]
        acc = acc + w * (_dot_t(h_expert, weo_v[e]) + beo_ref[e:e + 1, :])

    @pl.when(i == 0)
    def _wait_expert_out_hi():
        pltpu.make_async_copy(weo_hbm.at[EPG:], weo_v.at[EPG:], sem_o2).wait()

    for e in range(EPG, E):
        w = c8[:, e:e + 1]
        acc = acc + w * (_dot_t(h_expert, weo_v[e]) + beo_ref[e:e + 1, :])
    out_ref[...] = acc


def kernel(x, ln_w, ln_b, w_shared_in, w_shared_out, b_shared_out,
           w_expert_in, expert_out_w, expert_out_b,
           w_group_gate, w_expert_gate, group_bias, expert_bias):
    B, T, C = x.shape
    S = B * T
    G = w_group_gate.shape[0]
    E = expert_out_w.shape[0]
    EPG = E // G
    H = w_shared_out.shape[1]
    flat_x = x.reshape(S, C)

    const2 = lambda i: (0, 0)
    anyspec = pl.BlockSpec(memory_space=pl.ANY)
    out = pl.pallas_call(
        functools.partial(_fused, G, EPG, H),
        grid=(S // _BLK,),
        in_specs=[
            pl.BlockSpec((_BLK, C), lambda i: (i, 0)),
            pl.BlockSpec((1, C), const2),
            pl.BlockSpec((1, C), const2),
            pl.BlockSpec((1, C), const2),
            pl.BlockSpec((E, C), const2),
            pl.BlockSpec((G, C), const2),
            pl.BlockSpec((E, C), const2),
            pl.BlockSpec((1, G), const2),
            pl.BlockSpec((1, E), const2),
            pl.BlockSpec((C, H), const2),
            anyspec,
            anyspec,
            anyspec,
        ],
        out_specs=pl.BlockSpec((_BLK, C), lambda i: (i, 0)),
        out_shape=jax.ShapeDtypeStruct((S, C), jnp.float32),
        scratch_shapes=[
            pltpu.VMEM((2 * H, C), jnp.float32),
            pltpu.VMEM((2 * H, C), jnp.float32),
            pltpu.VMEM((E, C, H), jnp.float32),
            pltpu.SemaphoreType.DMA,
            pltpu.SemaphoreType.DMA,
            pltpu.SemaphoreType.DMA,
            pltpu.SemaphoreType.DMA,
        ],
        compiler_params=pltpu.CompilerParams(
            dimension_semantics=("arbitrary",),
            vmem_limit_bytes=128 * 1024 * 1024,
        ),
    )(flat_x, ln_w.reshape(1, C), ln_b.reshape(1, C),
      b_shared_out.reshape(1, C), expert_out_b, w_group_gate, w_expert_gate,
      group_bias.reshape(1, G), expert_bias.reshape(1, E), w_shared_out,
      w_shared_in, w_expert_in, expert_out_w)
    return out.reshape(B, T, C)


# speedup vs baseline: 1.1216x; 1.1216x over previous
import functools

import jax
import jax.numpy as jnp
from jax.experimental import pallas as pl
from jax.experimental.pallas import tpu as pltpu

_EPS = 1e-5
_TOPK = 2
_BLK = 512


def _dot_t(a, b, precision=None):
    return jax.lax.dot_general(
        a, b, dimension_numbers=(((1,), (1,)), ((), ())),
        precision=precision, preferred_element_type=jnp.float32)


def _silu(v):
    return v * jax.nn.sigmoid(v)


def _body(first, G, EPG, H, x_ref, lnw_ref, lnb_ref, bso_ref, beo_ref,
          wg_ref, we_ref, gb_ref, eb_ref, wso_ref, wsi_hbm, wei_hbm, weo_hbm,
          out_ref, wsi_v, wei_v, weo_v, sem_si, sem_ei, sem_o1, sem_o2):
    E = G * EPG
    if first:
        pltpu.make_async_copy(wsi_hbm, wsi_v, sem_si).start()
        pltpu.make_async_copy(wei_hbm, wei_v, sem_ei).start()
        pltpu.make_async_copy(weo_hbm.at[:EPG], weo_v.at[:EPG], sem_o1).start()
        pltpu.make_async_copy(weo_hbm.at[EPG:], weo_v.at[EPG:], sem_o2).start()

    xb = x_ref[...]
    mu = jnp.mean(xb, axis=-1, keepdims=True)
    xc = xb - mu
    var = jnp.mean(xc * xc, axis=-1, keepdims=True)
    flat = xc / jnp.sqrt(var + _EPS) * lnw_ref[...] + lnb_ref[...]

    g = _dot_t(flat, wg_ref[...]) + gb_ref[...]
    el = _dot_t(flat, we_ref[...])
    eb = eb_ref[...]

    gmask = g[:, 1:2] > g[:, 0:1]
    e4 = (jnp.where(gmask, el[:, EPG:], el[:, :EPG])
          + jnp.where(gmask, eb[:, EPG:], eb[:, :EPG]))
    m = jnp.max(e4, axis=-1, keepdims=True)
    ex = jnp.exp(e4 - m)
    p = ex / jnp.sum(ex, axis=-1, keepdims=True)

    cols = [p[:, c_:c_ + 1] for c_ in range(EPG)]
    keep = []
    for e in range(EPG):
        cnt = jnp.zeros_like(cols[0])
        for j in range(EPG):
            if j == e:
                continue
            beats = (cols[j] >= cols[e]) if j < e else (cols[j] > cols[e])
            cnt = cnt + beats.astype(jnp.float32)
        keep.append((cnt < float(_TOPK)).astype(jnp.float32))
    c4 = p * jnp.concatenate(keep, axis=-1)
    gm = gmask.astype(jnp.float32)
    c8 = jnp.concatenate([c4 * (1.0 - gm), c4 * gm], axis=-1)

    if first:
        pltpu.make_async_copy(wsi_hbm, wsi_v, sem_si).wait()

    h_shared = (_silu(_dot_t(flat, wsi_v[:H, :])) * _dot_t(flat, wsi_v[H:, :]))
    acc = _dot_t(h_shared, wso_ref[...]) + bso_ref[...]

    if first:
        pltpu.make_async_copy(wei_hbm, wei_v, sem_ei).wait()

    h_expert = (_silu(_dot_t(flat, wei_v[:H, :])) * _dot_t(flat, wei_v[H:, :]))

    if first:
        pltpu.make_async_copy(weo_hbm.at[:EPG], weo_v.at[:EPG], sem_o1).wait()

    for e in range(EPG):
        w = c8[:, e:e + 1]
        acc = acc + w * (_dot_t(h_expert, weo_v[e]) + beo_ref[e:e + 1, :])

    if first:
        pltpu.make_async_copy(weo_hbm.at[EPG:], weo_v.at[EPG:], sem_o2).wait()

    for e in range(EPG, E):
        w = c8[:, e:e + 1]
        acc = acc + w * (_dot_t(h_expert, weo_v[e]) + beo_ref[e:e + 1, :])
    out_ref[...] = acc


def _fused(G, EPG, H, *refs):
    i = pl.program_id(0)

    @pl.when(i == 0)
    def _first_step():
        _body(True, G, EPG, H, *refs)

    @pl.when(i != 0)
    def _steady_state():
        _body(False, G, EPG, H, *refs)


def kernel(x, ln_w, ln_b, w_shared_in, w_shared_out, b_shared_out,
           w_expert_in, expert_out_w, expert_out_b,
           w_group_gate, w_expert_gate, group_bias, expert_bias):
    B, T, C = x.shape
    S = B * T
    G = w_group_gate.shape[0]
    E = expert_out_w.shape[0]
    EPG = E // G
    H = w_shared_out.shape[1]
    flat_x = x.reshape(S, C)

    const2 = lambda i: (0, 0)
    anyspec = pl.BlockSpec(memory_space=pl.ANY)
    out = pl.pallas_call(
        functools.partial(_fused, G, EPG, H),
        grid=(S // _BLK,),
        in_specs=[
            pl.BlockSpec((_BLK, C), lambda i: (i, 0)),
            pl.BlockSpec((1, C), const2),
            pl.BlockSpec((1, C), const2),
            pl.BlockSpec((1, C), const2),
            pl.BlockSpec((E, C), const2),
            pl.BlockSpec((G, C), const2),
            pl.BlockSpec((E, C), const2),
            pl.BlockSpec((1, G), const2),
            pl.BlockSpec((1, E), const2),
            pl.BlockSpec((C, H), const2),
            anyspec,
            anyspec,
            anyspec,
        ],
        out_specs=pl.BlockSpec((_BLK, C), lambda i: (i, 0)),
        out_shape=jax.ShapeDtypeStruct((S, C), jnp.float32),
        scratch_shapes=[
            pltpu.VMEM((2 * H, C), jnp.float32),
            pltpu.VMEM((2 * H, C), jnp.float32),
            pltpu.VMEM((E, C, H), jnp.float32),
            pltpu.SemaphoreType.DMA,
            pltpu.SemaphoreType.DMA,
            pltpu.SemaphoreType.DMA,
            pltpu.SemaphoreType.DMA,
        ],
        compiler_params=pltpu.CompilerParams(
            dimension_semantics=("arbitrary",),
            vmem_limit_bytes=128 * 1024 * 1024,
        ),
    )(flat_x, ln_w.reshape(1, C), ln_b.reshape(1, C),
      b_shared_out.reshape(1, C), expert_out_b, w_group_gate, w_expert_gate,
      group_bias.reshape(1, G), expert_bias.reshape(1, E), w_shared_out,
      w_shared_in, w_expert_in, expert_out_w)
    return out.reshape(B, T, C)
